# Initial kernel scaffold; baseline (speedup 1.0000x reference)
import jax
import jax.numpy as jnp
from jax.experimental import pallas as pl


def kernel(indices, table):
    raise NotImplementedError("write your pallas kernel here")


# speedup vs baseline: 1.4713x; 1.4713x over previous
import functools

import jax
import jax.numpy as jnp
from jax import lax
from jax.experimental import pallas as pl
from jax.experimental.pallas import tpu as pltpu
from jax.experimental.pallas import tpu_sc as plsc

B = 16384
L = 50
LP = 52
D = 64
NC, NS = 2, 16
NW = NC * NS
PAIR = 2 * LP
NPAIR = B // 2
CP = 8
ROWS = CP * PAIR
CHUNKS = NPAIR // (NW * CP)
VL = 16


def _phrase_kernel(idx_hbm, table_hbm, out_hbm, idx_v, buf, outb, sem):
    wid = lax.axis_index("s") * NC + lax.axis_index("c")
    pair0 = wid * (NPAIR // NW)
    zero = jnp.zeros((VL,), jnp.float32)
    inv_l = jnp.float32(1.0 / L)

    def chunk(g, carry):
        pb = pair0 + g * CP
        pltpu.sync_copy(idx_hbm.at[pl.ds(pb, CP)], idx_v)
        copies = [
            pltpu.async_copy(
                table_hbm.at[idx_v.at[s]], buf.at[pl.ds(s * PAIR, PAIR)], sem
            )
            for s in range(CP)
        ]
        for c in copies:
            c.wait()
        for p in range(2 * CP):
            base = (p // 2) * PAIR + (p % 2) * LP

            def body(j, acc):
                row = base + j
                return tuple(
                    acc[c] + buf[row, pl.ds(c * VL, VL)] for c in range(D // VL)
                )

            acc = lax.fori_loop(0, L, body, (zero,) * (D // VL))
            for c in range(D // VL):
                outb[p, pl.ds(c * VL, VL)] = acc[c] * inv_l
        pltpu.sync_copy(outb, out_hbm.at[pl.ds(2 * pb, 2 * CP)])
        return carry

    lax.fori_loop(0, CHUNKS, chunk, 0)


def kernel(indices, table):
    idx = jnp.pad(indices.astype(jnp.int32), ((0, 0), (0, LP - L)))
    idx = idx.reshape(NPAIR, PAIR)
    mesh = plsc.VectorSubcoreMesh(core_axis_name="c", subcore_axis_name="s")
    run = functools.partial(
        pl.kernel,
        out_type=jax.ShapeDtypeStruct((B, D), jnp.float32),
        mesh=mesh,
        compiler_params=pltpu.CompilerParams(use_tc_tiling_on_sc=False),
        scratch_types=[
            pltpu.VMEM((CP, PAIR), jnp.int32),
            pltpu.VMEM((ROWS, D), jnp.float32),
            pltpu.VMEM((2 * CP, D), jnp.float32),
            pltpu.SemaphoreType.DMA,
        ],
    )(_phrase_kernel)
    return run(idx, table)


# speedup vs baseline: 1.4748x; 1.0024x over previous
import functools

import jax
import jax.numpy as jnp
from jax import lax
from jax.experimental import pallas as pl
from jax.experimental.pallas import tpu as pltpu
from jax.experimental.pallas import tpu_sc as plsc

B = 16384
L = 50
LP = 52
D = 64
NC, NS = 2, 16
NW = NC * NS
PAIR = 2 * LP
NPAIR = B // 2
CP = 8
ROWS = CP * PAIR
CHUNKS = NPAIR // (NW * CP)
VL = 16


def _phrase_kernel(idx_hbm, table_hbm, out_hbm, idx_v, buf, outb, sem):
    wid = lax.axis_index("s") * NC + lax.axis_index("c")
    pair0 = wid * (NPAIR // NW)
    zero = jnp.zeros((VL,), jnp.float32)
    inv_l = jnp.float32(1.0 / L)

    def chunk(g, carry):
        pb = pair0 + g * CP
        pltpu.sync_copy(idx_hbm.at[pl.ds(pb, CP)], idx_v)
        copies = [
            pltpu.async_copy(
                table_hbm.at[idx_v.at[s]], buf.at[pl.ds(s * PAIR, PAIR)], sem
            )
            for s in range(CP)
        ]
        for c in copies:
            c.wait()
        for p in range(2 * CP):
            base = (p // 2) * PAIR + (p % 2) * LP

            def body(j, acc):
                row = base + j
                return tuple(
                    acc[c] + buf[row, pl.ds(c * VL, VL)] for c in range(D // VL)
                )

            acc = lax.fori_loop(0, L, body, (zero,) * (D // VL), unroll=10)
            for c in range(D // VL):
                outb[p, pl.ds(c * VL, VL)] = acc[c] * inv_l
        pltpu.sync_copy(outb, out_hbm.at[pl.ds(2 * pb, 2 * CP)])
        return carry

    lax.fori_loop(0, CHUNKS, chunk, 0)


def kernel(indices, table):
    idx = jnp.pad(indices.astype(jnp.int32), ((0, 0), (0, LP - L)))
    idx = idx.reshape(NPAIR, PAIR)
    mesh = plsc.VectorSubcoreMesh(core_axis_name="c", subcore_axis_name="s")
    run = functools.partial(
        pl.kernel,
        out_type=jax.ShapeDtypeStruct((B, D), jnp.float32),
        mesh=mesh,
        compiler_params=pltpu.CompilerParams(use_tc_tiling_on_sc=False),
        scratch_types=[
            pltpu.VMEM((CP, PAIR), jnp.int32),
            pltpu.VMEM((ROWS, D), jnp.float32),
            pltpu.VMEM((2 * CP, D), jnp.float32),
            pltpu.SemaphoreType.DMA,
        ],
    )(_phrase_kernel)
    return run(idx, table)


# speedup vs baseline: 2.4878x; 1.6869x over previous
import functools

import jax
import jax.numpy as jnp
from jax import lax
from jax.experimental import pallas as pl
from jax.experimental.pallas import tpu as pltpu
from jax.experimental.pallas import tpu_sc as plsc

B = 16384
L = 50
D = 64
NC, NS = 2, 16
NW = NC * NS
PAIR = 2 * L
NPAIR = B // 2
CP = 8
CHUNKS = NPAIR // (NW * CP)
STRIDE = 104
VL = 16


def _phrase_kernel(idx_hbm, table_hbm, out_hbm, idx_v, buf, outb, sem):
    wid = lax.axis_index("s") * NC + lax.axis_index("c")
    pair0 = wid * (NPAIR // NW)
    zero = jnp.zeros((VL,), jnp.float32)
    inv_l = jnp.float32(1.0 / L)

    def chunk(g, carry):
        pb = pair0 + g * CP
        pltpu.sync_copy(idx_hbm.at[pl.ds(pb, CP)], idx_v)
        copies = [
            pltpu.async_copy(
                table_hbm.at[idx_v.at[s]], buf.at[pl.ds(s * STRIDE, PAIR)], sem
            )
            for s in range(CP)
        ]
        for c in copies:
            c.wait()
        for p in range(2 * CP):
            base = (p // 2) * STRIDE + (p % 2) * L

            def body(j, acc):
                return tuple(
                    acc[c] + buf[base + j, pl.ds(c * VL, VL)]
                    for c in range(D // VL)
                )

            acc = lax.fori_loop(0, L, body, (zero,) * (D // VL), unroll=10)
            for c in range(D // VL):
                outb[p, pl.ds(c * VL, VL)] = acc[c] * inv_l
        pltpu.sync_copy(outb, out_hbm.at[pl.ds(2 * pb, 2 * CP)])
        return carry

    lax.fori_loop(0, CHUNKS, chunk, 0)


def kernel(indices, table):
    idx = indices.astype(jnp.int32).reshape(NPAIR, PAIR)
    mesh = plsc.VectorSubcoreMesh(core_axis_name="c", subcore_axis_name="s")
    run = functools.partial(
        pl.kernel,
        out_type=jax.ShapeDtypeStruct((B, D), jnp.float32),
        mesh=mesh,
        compiler_params=pltpu.CompilerParams(use_tc_tiling_on_sc=False),
        scratch_types=[
            pltpu.VMEM((CP, PAIR), jnp.int32),
            pltpu.VMEM((CP * STRIDE, D), jnp.float32),
            pltpu.VMEM((2 * CP, D), jnp.float32),
            pltpu.SemaphoreType.DMA,
        ],
    )(_phrase_kernel)
    return run(idx, table)


# speedup vs baseline: 2.7006x; 1.0855x over previous
import functools

import jax
import jax.numpy as jnp
from jax import lax
from jax.experimental import pallas as pl
from jax.experimental.pallas import tpu as pltpu
from jax.experimental.pallas import tpu_sc as plsc

B = 16384
L = 50
D = 64
NC, NS = 2, 16
NW = NC * NS
CB = 16
ROWS = CB * L
CHUNKS = B // (NW * CB)
VL = 16
NIDX = 128
NST = ROWS // NIDX
REM = ROWS - NST * NIDX


def _phrase_kernel(idx_hbm, table_hbm, out_hbm, idx0, idx1, buf0, buf1,
                   outb, sem0, sem1):
    wid = lax.axis_index("s") * NC + lax.axis_index("c")
    chunk0 = wid * CHUNKS
    zero = jnp.zeros((VL,), jnp.float32)
    inv_l = jnp.float32(1.0 / L)

    def fire(g, idx_v, sem, buf):
        pltpu.sync_copy(idx_hbm.at[pl.ds(g * ROWS, ROWS)], idx_v)
        for s in range(NST):
            pltpu.async_copy(
                table_hbm.at[idx_v.at[pl.ds(s * NIDX, NIDX)]],
                buf.at[pl.ds(s * NIDX, NIDX)], sem)
        if REM:
            pltpu.async_copy(
                table_hbm.at[idx_v.at[pl.ds(NST * NIDX, REM)]],
                buf.at[pl.ds(NST * NIDX, REM)], sem)

    def drain(idx_v, sem, buf):
        pltpu.make_async_copy(table_hbm.at[idx_v], buf, sem).wait()

    def reduce(g, buf):
        for p in range(CB):
            base = p * L

            def body(j, acc):
                return tuple(
                    acc[c] + buf[base + j, pl.ds(c * VL, VL)]
                    for c in range(D // VL)
                )

            acc = lax.fori_loop(0, L, body, (zero,) * (D // VL), unroll=10)
            for c in range(D // VL):
                outb[p, pl.ds(c * VL, VL)] = acc[c] * inv_l
        pltpu.sync_copy(outb, out_hbm.at[pl.ds(g * CB, CB)])

    fire(chunk0, idx0, sem0, buf0)

    def step(t, carry):
        g = chunk0 + 2 * t
        fire(g + 1, idx1, sem1, buf1)
        drain(idx0, sem0, buf0)
        reduce(g, buf0)

        @pl.when(t < CHUNKS // 2 - 1)
        def _():
            fire(g + 2, idx0, sem0, buf0)

        drain(idx1, sem1, buf1)
        reduce(g + 1, buf1)
        return carry

    lax.fori_loop(0, CHUNKS // 2, step, 0)


def kernel(indices, table):
    idx = indices.astype(jnp.int32).reshape(B * L)
    mesh = plsc.VectorSubcoreMesh(core_axis_name="c", subcore_axis_name="s")
    run = functools.partial(
        pl.kernel,
        out_type=jax.ShapeDtypeStruct((B, D), jnp.float32),
        mesh=mesh,
        compiler_params=pltpu.CompilerParams(use_tc_tiling_on_sc=False),
        scratch_types=[
            pltpu.VMEM((ROWS,), jnp.int32),
            pltpu.VMEM((ROWS,), jnp.int32),
            pltpu.VMEM((ROWS, D), jnp.float32),
            pltpu.VMEM((ROWS, D), jnp.float32),
            pltpu.VMEM((CB, D), jnp.float32),
            pltpu.SemaphoreType.DMA,
            pltpu.SemaphoreType.DMA,
        ],
    )(_phrase_kernel)
    return run(idx, table)
